# baseline (speedup 1.0000x reference)
import functools

import jax
import jax.numpy as jnp
import numpy as np
from jax import lax
from jax.experimental import pallas as pl
from jax.experimental.pallas import tpu as pltpu
from jax.experimental.pallas import tpu_sc as plsc

_K = 64
_T_MAX = 10080.0
_D_MAX = 200.0
_SCALE_T = np.float32((_K - 1) / _T_MAX)
_SCALE_D = np.float32((_K - 1) / _D_MAX)
_CLIP = np.float32(_K - 1 - 1e-06)

_NC, _NS, _L = 2, 16, 16
_NW = _NC * _NS
_B, _S = 4, 2048
_RPW = _B * _S // _NW
_SLAB = 8
_G = _RPW // _SLAB
_VROW = _S // _L


def _compute_slab(dt_v, dd_v, out_v, et_v, det_v, ed_v, ded_v):
    @plsc.parallel_loop(0, _SLAB * _VROW, unroll=8)
    def vbody(i):
        r = i // _VROW
        off = pl.multiple_of((i % _VROW) * _L, _L)
        t = dt_v[r, pl.ds(off, _L)]
        d = dd_v[r, pl.ds(off, _L)]
        tb = jnp.minimum(jnp.maximum(t, 0.0) * _SCALE_T, _CLIP)
        db = jnp.minimum(jnp.maximum(d, 0.0) * _SCALE_D, _CLIP)
        kt = tb.astype(jnp.int32)
        kd = db.astype(jnp.int32)
        a_t = plsc.load_gather(et_v, [kt])
        c_t = plsc.load_gather(det_v, [kt])
        a_d = plsc.load_gather(ed_v, [kd])
        c_d = plsc.load_gather(ded_v, [kd])
        out_v[r, pl.ds(off, _L)] = (a_t + tb * c_t) + (a_d + db * c_d)


def _sc_body(dt_hbm, dd_hbm, tab_hbm, out_hbm,
             et_v, det_v, ed_v, ded_v, dt_v0, dt_v1, dd_v0, dd_v1,
             out_v0, out_v1, isem0, isem1, osem0, osem1):
    wid = lax.axis_index("s") * _NC + lax.axis_index("c")
    b = wid // (_S // _RPW)
    r0 = (wid % (_S // _RPW)) * _RPW
    pltpu.sync_copy(tab_hbm.at[pl.ds(0, _K)], et_v)
    pltpu.sync_copy(tab_hbm.at[pl.ds(_K, _K)], det_v)
    pltpu.sync_copy(tab_hbm.at[pl.ds(2 * _K, _K)], ed_v)
    pltpu.sync_copy(tab_hbm.at[pl.ds(3 * _K, _K)], ded_v)

    bufs = ((dt_v0, dd_v0, out_v0, isem0, osem0),
            (dt_v1, dd_v1, out_v1, isem1, osem1))

    def start_in(g, bi):
        dtv, ddv, _, isem, _ = bufs[bi]
        r = r0 + g * _SLAB
        pltpu.make_async_copy(
            dt_hbm.at[b, pl.ds(r, _SLAB), :], dtv, isem).start()
        pltpu.make_async_copy(
            dd_hbm.at[b, pl.ds(r, _SLAB), :], ddv, isem).start()

    def wait_in(bi):
        dtv, ddv, _, isem, _ = bufs[bi]
        pltpu.make_async_copy(
            dt_hbm.at[0, pl.ds(0, _SLAB), :], dtv, isem).wait()
        pltpu.make_async_copy(
            dd_hbm.at[0, pl.ds(0, _SLAB), :], ddv, isem).wait()

    def start_out(g, bi):
        _, _, ov, _, osem = bufs[bi]
        r = r0 + g * _SLAB
        pltpu.make_async_copy(
            ov, out_hbm.at[b, pl.ds(r, _SLAB), :], osem).start()

    def wait_out(bi):
        _, _, ov, _, osem = bufs[bi]
        pltpu.make_async_copy(
            ov, out_hbm.at[0, pl.ds(0, _SLAB), :], osem).wait()

    start_in(0, 0)
    start_in(1, 1)

    def ubody(u, carry):
        for bi in range(2):
            g = u * 2 + bi
            dtv, ddv, ov, _, _ = bufs[bi]
            wait_in(bi)

            @pl.when(g >= 2)
            def _():
                wait_out(bi)

            _compute_slab(dtv, ddv, ov, et_v, det_v, ed_v, ded_v)
            start_out(g, bi)

            @pl.when(g + 2 < _G)
            def _():
                start_in(g + 2, bi)
        return carry

    lax.fori_loop(0, _G // 2, ubody, 0)
    wait_out(0)
    wait_out(1)


_sc_call = functools.partial(
    pl.kernel,
    out_type=jax.ShapeDtypeStruct((_B, _S, _S), jnp.float32),
    mesh=plsc.VectorSubcoreMesh(
        core_axis_name="c", subcore_axis_name="s",
        num_cores=_NC, num_subcores=_NS),
    scratch_types=[
        pltpu.VMEM((_K,), jnp.float32),
        pltpu.VMEM((_K,), jnp.float32),
        pltpu.VMEM((_K,), jnp.float32),
        pltpu.VMEM((_K,), jnp.float32),
        pltpu.VMEM((_SLAB, _S), jnp.float32),
        pltpu.VMEM((_SLAB, _S), jnp.float32),
        pltpu.VMEM((_SLAB, _S), jnp.float32),
        pltpu.VMEM((_SLAB, _S), jnp.float32),
        pltpu.VMEM((_SLAB, _S), jnp.float32),
        pltpu.VMEM((_SLAB, _S), jnp.float32),
        pltpu.SemaphoreType.DMA,
        pltpu.SemaphoreType.DMA,
        pltpu.SemaphoreType.DMA,
        pltpu.SemaphoreType.DMA,
    ],
    compiler_params=pltpu.CompilerParams(
        needs_layout_passes=False, use_tc_tiling_on_sc=True,
        disable_bounds_checks=True, disable_semaphore_checks=True),
)(_sc_body)


def kernel(dt_minutes, dd_km, E_t, E_d):
    zero = jnp.zeros((1,), jnp.float32)
    d_t = jnp.concatenate([E_t[1:] - E_t[:-1], zero])
    d_d = jnp.concatenate([E_d[1:] - E_d[:-1], zero])
    karange = jnp.arange(_K, dtype=jnp.float32)
    tab = jnp.concatenate(
        [E_t - karange * d_t, d_t, E_d - karange * d_d, d_d])
    return _sc_call(dt_minutes, dd_km, tab)


# speedup vs baseline: 1.0023x; 1.0023x over previous
import functools

import jax
import jax.numpy as jnp
import numpy as np
from jax import lax
from jax.experimental import pallas as pl
from jax.experimental.pallas import tpu as pltpu
from jax.experimental.pallas import tpu_sc as plsc

_K = 64
_T_MAX = 10080.0
_D_MAX = 200.0
_SCALE_T = np.float32((_K - 1) / _T_MAX)
_SCALE_D = np.float32((_K - 1) / _D_MAX)
_CLIP = np.float32(_K - 1 - 1e-06)

_NC, _NS, _L = 2, 16, 16
_NW = _NC * _NS
_B, _S = 4, 2048
_RPW = _B * _S // _NW
_SLAB = 8
_G = _RPW // _SLAB
_VROW = _S // _L


def _compute_slab(dt_v, dd_v, out_v, et_v, det_v, ed_v, ded_v):
    @plsc.parallel_loop(0, _SLAB * _VROW, unroll=8)
    def vbody(i):
        r = i // _VROW
        off = pl.multiple_of((i % _VROW) * _L, _L)
        t = dt_v[r, pl.ds(off, _L)]
        d = dd_v[r, pl.ds(off, _L)]
        tb = jnp.minimum(jnp.maximum(t, 0.0) * _SCALE_T, _CLIP)
        db = jnp.minimum(jnp.maximum(d, 0.0) * _SCALE_D, _CLIP)
        kt = tb.astype(jnp.int32)
        kd = db.astype(jnp.int32)
        a_t = plsc.load_gather(et_v, [kt])
        c_t = plsc.load_gather(det_v, [kt])
        a_d = plsc.load_gather(ed_v, [kd])
        c_d = plsc.load_gather(ded_v, [kd])
        out_v[r, pl.ds(off, _L)] = (a_t + tb * c_t) + (a_d + db * c_d)


def _sc_body(dt_hbm, dd_hbm, tab_hbm, out_hbm,
             et_v, det_v, ed_v, ded_v, dt_v0, dt_v1, dd_v0, dd_v1,
             out_v0, out_v1, isem0, isem1, osem0, osem1):
    wid = lax.axis_index("s") * _NC + lax.axis_index("c")
    b = wid // (_S // _RPW)
    r0 = (wid % (_S // _RPW)) * _RPW
    pltpu.sync_copy(tab_hbm.at[pl.ds(0, _K)], et_v)
    pltpu.sync_copy(tab_hbm.at[pl.ds(_K, _K)], det_v)
    pltpu.sync_copy(tab_hbm.at[pl.ds(2 * _K, _K)], ed_v)
    pltpu.sync_copy(tab_hbm.at[pl.ds(3 * _K, _K)], ded_v)

    bufs = ((dt_v0, dd_v0, out_v0, isem0, osem0),
            (dt_v1, dd_v1, out_v1, isem1, osem1))

    def start_in(g, bi):
        dtv, ddv, _, isem, _ = bufs[bi]
        r = r0 + g * _SLAB
        pltpu.make_async_copy(
            dt_hbm.at[b, pl.ds(r, _SLAB), :], dtv, isem).start()
        pltpu.make_async_copy(
            dd_hbm.at[b, pl.ds(r, _SLAB), :], ddv, isem).start()

    def wait_in(bi):
        dtv, ddv, _, isem, _ = bufs[bi]
        pltpu.make_async_copy(
            dt_hbm.at[0, pl.ds(0, _SLAB), :], dtv, isem).wait()
        pltpu.make_async_copy(
            dd_hbm.at[0, pl.ds(0, _SLAB), :], ddv, isem).wait()

    def start_out(g, bi):
        _, _, ov, _, osem = bufs[bi]
        r = r0 + g * _SLAB
        pltpu.make_async_copy(
            ov, out_hbm.at[b, pl.ds(r, _SLAB), :], osem).start()

    def wait_out(bi):
        _, _, ov, _, osem = bufs[bi]
        pltpu.make_async_copy(
            ov, out_hbm.at[0, pl.ds(0, _SLAB), :], osem).wait()

    start_in(0, 0)
    start_in(1, 1)

    def ubody(u, carry):
        for bi in range(2):
            g = u * 2 + bi
            dtv, ddv, ov, _, _ = bufs[bi]
            wait_in(bi)

            @pl.when(g >= 2)
            def _():
                wait_out(bi)

            _compute_slab(dtv, ddv, ov, et_v, det_v, ed_v, ded_v)
            start_out(g, bi)

            @pl.when(g + 2 < _G)
            def _():
                start_in(g + 2, bi)
        return carry

    lax.fori_loop(0, _G // 2, ubody, 0)
    wait_out(0)
    wait_out(1)


_sc_call = functools.partial(
    pl.kernel,
    out_type=jax.ShapeDtypeStruct((_B, _S, _S), jnp.float32),
    mesh=plsc.VectorSubcoreMesh(
        core_axis_name="c", subcore_axis_name="s",
        num_cores=_NC, num_subcores=_NS),
    scratch_types=[
        pltpu.VMEM((_K,), jnp.float32),
        pltpu.VMEM((_K,), jnp.float32),
        pltpu.VMEM((_K,), jnp.float32),
        pltpu.VMEM((_K,), jnp.float32),
        pltpu.VMEM((_SLAB, _S), jnp.float32),
        pltpu.VMEM((_SLAB, _S), jnp.float32),
        pltpu.VMEM((_SLAB, _S), jnp.float32),
        pltpu.VMEM((_SLAB, _S), jnp.float32),
        pltpu.VMEM((_SLAB, _S), jnp.float32),
        pltpu.VMEM((_SLAB, _S), jnp.float32),
        pltpu.SemaphoreType.DMA,
        pltpu.SemaphoreType.DMA,
        pltpu.SemaphoreType.DMA,
        pltpu.SemaphoreType.DMA,
    ],
    compiler_params=pltpu.CompilerParams(
        needs_layout_passes=False, use_tc_tiling_on_sc=True),
)(_sc_body)


def kernel(dt_minutes, dd_km, E_t, E_d):
    zero = jnp.zeros((1,), jnp.float32)
    d_t = jnp.concatenate([E_t[1:] - E_t[:-1], zero])
    d_d = jnp.concatenate([E_d[1:] - E_d[:-1], zero])
    karange = jnp.arange(_K, dtype=jnp.float32)
    tab = jnp.concatenate(
        [E_t - karange * d_t, d_t, E_d - karange * d_d, d_d])
    return _sc_call(dt_minutes, dd_km, tab)
